# Initial kernel scaffold; baseline (speedup 1.0000x reference)
import jax
import jax.numpy as jnp
from jax.experimental import pallas as pl


def kernel(x, edge_index, W_h, b_h, W_mu, b_mu, W_ls, b_ls):
    raise NotImplementedError("write your pallas kernel here")


# speedup vs baseline: 14.2640x; 14.2640x over previous
import jax
import jax.numpy as jnp
from jax import lax
from jax.experimental import pallas as pl
from jax.experimental.pallas import tpu as pltpu
from jax.experimental.pallas import tpu_sc as plsc

N = 10000
E = 320000
D = 128
Z = 64
NC = 2
NS = 16
NW = NC * NS
EPW = E // NW
K = 128
CH = 79
EPW2 = CH * K
PADE = EPW2 - EPW
NACC = 10016
NPAD = 10240
DPS = NPAD // NS
RS = 624
TAIL = N - NS * RS

_mesh = plsc.VectorSubcoreMesh(core_axis_name="c", subcore_axis_name="s")


def _deg_body(dst_hbm, deg_out, didx_c, ones_v, zbuf_v, degacc):
    c = lax.axis_index("c")
    s = lax.axis_index("s")
    wid = s * NC + c
    for k in range(K // 16):
        ones_v[pl.ds(16 * k, 16)] = jnp.full((16,), 1.0, jnp.float32)
    for k in range(DPS // 16):
        zbuf_v[pl.ds(16 * k, 16)] = jnp.zeros((16,), jnp.float32)
    pltpu.sync_copy(zbuf_v, degacc.at[pl.ds(s * DPS, DPS)])
    plsc.subcore_barrier()

    def chunk(j, carry):
        off = pl.multiple_of(wid * EPW2 + j * K, K)
        pltpu.sync_copy(dst_hbm.at[pl.ds(off, K)], didx_c)
        pltpu.sync_copy(ones_v, degacc.at[didx_c], add=True)
        return carry

    lax.fori_loop(0, CH, chunk, 0)
    plsc.subcore_barrier()
    pltpu.sync_copy(degacc.at[pl.ds(s * DPS, DPS)],
                    deg_out.at[pl.ds(c * NPAD + s * DPS, DPS)])


def _sc_deg(dst_flat):
    return pl.kernel(
        _deg_body,
        out_type=jax.ShapeDtypeStruct((NC * NPAD,), jnp.float32),
        mesh=_mesh,
        scratch_types=[
            pltpu.VMEM((K,), jnp.int32),
            pltpu.VMEM((K,), jnp.float32),
            pltpu.VMEM((DPS,), jnp.float32),
            pltpu.VMEM_SHARED((NPAD,), jnp.float32),
        ],
    )(dst_flat)


def _agg_body(src_hbm, dst_hbm, y_hbm, out_hbm, sidx_c, didx_c, rows_v, acc,
              sem):
    c = lax.axis_index("c")
    s = lax.axis_index("s")
    wid = s * NC + c
    pltpu.sync_copy(y_hbm.at[pl.ds(s * RS, RS)], acc.at[pl.ds(s * RS, RS)])

    @pl.when(s == 0)
    def _():
        pltpu.sync_copy(y_hbm.at[pl.ds(NS * RS, TAIL)],
                        acc.at[pl.ds(NS * RS, TAIL)])

    plsc.subcore_barrier()

    def chunk(j, carry):
        off = pl.multiple_of(wid * EPW2 + j * K, K)
        pltpu.sync_copy(src_hbm.at[pl.ds(off, K)], sidx_c)
        pltpu.sync_copy(dst_hbm.at[pl.ds(off, K)], didx_c)
        pltpu.async_copy(y_hbm.at[sidx_c], rows_v, sem).wait()
        pltpu.sync_copy(rows_v, acc.at[didx_c], add=True)
        return carry

    lax.fori_loop(0, CH, chunk, 0)
    plsc.subcore_barrier()
    pltpu.sync_copy(acc.at[pl.ds(s * RS, RS)],
                    out_hbm.at[c, pl.ds(s * RS, RS)])

    @pl.when(s == 0)
    def _():
        pltpu.sync_copy(acc.at[pl.ds(NS * RS, TAIL)],
                        out_hbm.at[c, pl.ds(NS * RS, TAIL)])


def _sc_agg(src_flat, dst_flat, y):
    return pl.kernel(
        _agg_body,
        out_type=jax.ShapeDtypeStruct((NC, N, D), jnp.float32),
        mesh=_mesh,
        scratch_types=[
            pltpu.VMEM((K,), jnp.int32),
            pltpu.VMEM((K,), jnp.int32),
            pltpu.VMEM((K, D), jnp.float32),
            pltpu.VMEM_SHARED((NACC, D), jnp.float32),
            pltpu.SemaphoreType.DMA,
        ],
    )(src_flat, dst_flat, y)


B = 2000


def _m1_body(degT_ref, x_ref, w_ref, y1_ref, dinv_ref):
    deg = degT_ref[:, 0:1] + degT_ref[:, 1:2] + 1.0
    dinv = lax.rsqrt(deg)
    dinv_ref[...] = dinv
    xw = jnp.dot(x_ref[...], w_ref[...], preferred_element_type=jnp.float32,
                 precision=lax.Precision.HIGHEST)
    y1_ref[...] = xw * dinv


def _m1(degT, x, W):
    return pl.pallas_call(
        _m1_body,
        grid=(N // B,),
        in_specs=[
            pl.BlockSpec((B, 2), lambda i: (i, 0)),
            pl.BlockSpec((B, D), lambda i: (i, 0)),
            pl.BlockSpec((D, D), lambda i: (0, 0)),
        ],
        out_specs=[
            pl.BlockSpec((B, D), lambda i: (i, 0)),
            pl.BlockSpec((B, 1), lambda i: (i, 0)),
        ],
        out_shape=[
            jax.ShapeDtypeStruct((N, D), jnp.float32),
            jax.ShapeDtypeStruct((N, 1), jnp.float32),
        ],
    )(degT, x, W)


def _m2_body(p_ref, y1_ref, dinv_ref, bh_ref, wcat_ref, y2_ref):
    dinv = dinv_ref[...]
    pre = (p_ref[0] + p_ref[1] - y1_ref[...]) * dinv + bh_ref[...]
    h = jnp.maximum(pre, 0.0)
    y2_ref[...] = jnp.dot(h, wcat_ref[...], preferred_element_type=jnp.float32,
                          precision=lax.Precision.HIGHEST) * dinv


def _m2(P, y1, dinv, bh, Wcat):
    return pl.pallas_call(
        _m2_body,
        grid=(N // B,),
        in_specs=[
            pl.BlockSpec((NC, B, D), lambda i: (0, i, 0)),
            pl.BlockSpec((B, D), lambda i: (i, 0)),
            pl.BlockSpec((B, 1), lambda i: (i, 0)),
            pl.BlockSpec((1, D), lambda i: (0, 0)),
            pl.BlockSpec((D, D), lambda i: (0, 0)),
        ],
        out_specs=pl.BlockSpec((B, D), lambda i: (i, 0)),
        out_shape=jax.ShapeDtypeStruct((N, D), jnp.float32),
    )(P, y1, dinv, bh, Wcat)


def _m3_body(q_ref, y2_ref, dinv_ref, bcat_ref, out_ref):
    out_ref[...] = ((q_ref[0] + q_ref[1] - y2_ref[...]) * dinv_ref[...]
                    + bcat_ref[...])


def _m3(Q, y2, dinv, bcat):
    return pl.pallas_call(
        _m3_body,
        grid=(N // B,),
        in_specs=[
            pl.BlockSpec((NC, B, D), lambda i: (0, i, 0)),
            pl.BlockSpec((B, D), lambda i: (i, 0)),
            pl.BlockSpec((B, 1), lambda i: (i, 0)),
            pl.BlockSpec((1, D), lambda i: (0, 0)),
        ],
        out_specs=pl.BlockSpec((B, D), lambda i: (i, 0)),
        out_shape=jax.ShapeDtypeStruct((N, D), jnp.float32),
    )(Q, y2, dinv, bcat)


def kernel(x, edge_index, W_h, b_h, W_mu, b_mu, W_ls, b_ls):
    src = edge_index[0].reshape(NW, EPW)
    dst = edge_index[1].reshape(NW, EPW)
    pad_s = jnp.zeros((NW, PADE), jnp.int32)
    pad_d = jnp.full((NW, PADE), N, jnp.int32)
    src_flat = jnp.concatenate([src, pad_s], axis=1).reshape(-1)
    dst_flat = jnp.concatenate([dst, pad_d], axis=1).reshape(-1)
    W_cat = jnp.concatenate([W_mu, W_ls], axis=1)
    b_cat = jnp.concatenate([b_mu, b_ls])[None, :]

    deg_flat = _sc_deg(dst_flat)
    degT = deg_flat.reshape(NC, NPAD)[:, :N].T
    y1, dinv = _m1(degT, x, W_h)
    P = _sc_agg(src_flat, dst_flat, y1)
    y2 = _m2(P, y1, dinv, b_h[None, :], W_cat)
    Q = _sc_agg(src_flat, dst_flat, y2)
    out = _m3(Q, y2, dinv, b_cat)
    return out[:, :Z], out[:, Z:]
